# baseline (speedup 1.0000x reference)
import functools

import jax
import jax.numpy as jnp
from jax import lax
from jax.experimental import pallas as pl
from jax.experimental.pallas import tpu as pltpu
from jax.experimental.pallas import tpu_sc as plsc

NC = 2
NS = 16
NW = NC * NS

IDX_ROW = 128
ROWS_PER_CHUNK = 4
CHUNK = IDX_ROW * ROWS_PER_CHUNK


@functools.lru_cache(maxsize=None)
def _make_gather(n_entity, d, b_total):
    assert b_total % (NW * CHUNK) == 0
    b_per_w = b_total // NW
    n_chunk = b_per_w // CHUNK
    irow_per_w = b_per_w // IDX_ROW

    mesh = plsc.VectorSubcoreMesh(core_axis_name="c", subcore_axis_name="s")

    @functools.partial(
        pl.kernel,
        mesh=mesh,
        out_type=jax.ShapeDtypeStruct((b_total, 2, d), jnp.float32),
        scratch_types=[
            pltpu.VMEM((ROWS_PER_CHUNK, IDX_ROW), jnp.int32),
            pltpu.VMEM((CHUNK, d), jnp.float32),
            pltpu.VMEM((CHUNK, d), jnp.float32),
            pltpu.SemaphoreType.DMA,
        ],
        compiler_params=pltpu.CompilerParams(use_tc_tiling_on_sc=False),
    )
    def k(table_hbm, idx_hbm, out_hbm, idx_v, rows_v, zeros_v, sem):
        wid = lax.axis_index("s") * NC + lax.axis_index("c")
        w_base = wid * b_per_w
        w_irow = wid * irow_per_w

        def zrow(r, _):
            for j in range(d // 16):
                zeros_v[r, pl.ds(j * 16, 16)] = jnp.zeros((16,), jnp.float32)
            return 0
        lax.fori_loop(0, CHUNK, zrow, 0)

        def body(ci, _):
            irow = w_irow + ci * ROWS_PER_CHUNK
            pltpu.sync_copy(idx_hbm.at[pl.ds(irow, ROWS_PER_CHUNK)], idx_v)
            descs = [
                pltpu.make_async_copy(
                    table_hbm.at[idx_v.at[j]],
                    rows_v.at[pl.ds(j * IDX_ROW, IDX_ROW)],
                    sem,
                )
                for j in range(ROWS_PER_CHUNK)
            ]
            for dsc in descs:
                dsc.start()
            cbase = w_base + ci * CHUNK
            pltpu.sync_copy(zeros_v, out_hbm.at[pl.ds(cbase, CHUNK), 1])
            for dsc in descs:
                dsc.wait()
            pltpu.sync_copy(rows_v, out_hbm.at[pl.ds(cbase, CHUNK), 0])
            return 0

        lax.fori_loop(0, n_chunk, body, 0)

    return k


def kernel(entity_table, entity_ids):
    n_entity, d = entity_table.shape
    nb, nk = entity_ids.shape
    b_total = nb * nk
    ids2 = entity_ids.reshape(b_total // IDX_ROW, IDX_ROW)
    out = _make_gather(n_entity, d, b_total)(entity_table, ids2)
    return out.reshape(nb, nk, 2 * d)
